# baseline (speedup 1.0000x reference)
import functools

import jax
import jax.numpy as jnp
from jax import lax
from jax.experimental import pallas as pl
from jax.experimental.pallas import tpu as pltpu
from jax.experimental.pallas import tpu_sc as plsc

NUM_HEADS = 9
HIST = 200
D = 32
NC = 2
NS = 16
NW = NC * NS
ROWS_PAD = 2048
RPW = ROWS_PAD // NW


def _sc_gather(item_blocks, idx8_pad, user_blocks, uidx8):
    mesh = plsc.VectorSubcoreMesh(
        core_axis_name="c", subcore_axis_name="s", num_cores=NC, num_subcores=NS
    )

    @functools.partial(
        pl.kernel,
        out_type=(
            jax.ShapeDtypeStruct((ROWS_PAD, 8, D), jnp.float32),
            jax.ShapeDtypeStruct((1, 8, D), jnp.float32),
        ),
        mesh=mesh,
        compiler_params=pltpu.CompilerParams(use_tc_tiling_on_sc=False),
        scratch_types=[
            pltpu.VMEM((RPW,), jnp.int32),
            pltpu.VMEM((RPW, 8, D), jnp.float32),
            pltpu.VMEM((8,), jnp.int32),
            pltpu.VMEM((8, 8, D), jnp.float32),
            pltpu.SemaphoreType.DMA,
        ],
    )
    def k(items_hbm, idx_hbm, users_hbm, uidx_hbm, out_hbm, uout_hbm,
          idx_v, blocks_v, uidx_v, ublk_v, sem):
        wid = lax.axis_index("s") * NC + lax.axis_index("c")
        base = wid * RPW
        pltpu.sync_copy(idx_hbm.at[pl.ds(base, RPW)], idx_v)
        pltpu.async_copy(items_hbm.at[idx_v], blocks_v, sem).wait()
        pltpu.sync_copy(blocks_v, out_hbm.at[pl.ds(base, RPW)])

        @pl.when(wid == 0)
        def _():
            pltpu.sync_copy(uidx_hbm, uidx_v)
            pltpu.async_copy(users_hbm.at[uidx_v], ublk_v, sem).wait()
            pltpu.sync_copy(ublk_v.at[pl.ds(0, 1)], uout_hbm)

    return k(item_blocks, idx8_pad, user_blocks, uidx8)


def _tc_dense_body(blk_ref, oh_ref, ublk_ref, uoh_ref, w1_ref, b1_ref,
                   w2_ref, b2_ref, fw1_ref, fb1_ref, fw2_ref, fb2_ref,
                   out_ref):
    ue_row = jnp.sum(ublk_ref[0] * uoh_ref[0], axis=0, keepdims=True)
    ue = jnp.broadcast_to(ue_row, (HIST, D))
    outs = []
    for i in range(NUM_HEADS):
        blk = blk_ref[i * HIST:(i + 1) * HIST]
        oh = oh_ref[i * HIST:(i + 1) * HIST]
        area = jnp.sum(blk * oh, axis=1)
        x = jnp.concatenate([ue, area], axis=1)
        h = jnp.maximum(
            jnp.dot(x, w1_ref[i], preferred_element_type=jnp.float32)
            + b1_ref[i:i + 1, :], 0.0)
        o = (jnp.dot(h, w2_ref[i], preferred_element_type=jnp.float32)
             + b2_ref[i:i + 1, :])
        m = jnp.max(o, axis=0, keepdims=True)
        e = jnp.exp(o - m)
        s = jnp.sum(e, axis=0, keepdims=True)
        outs.append(jnp.sum((e / s) * area, axis=0, keepdims=True))
    area_all = jnp.concatenate(outs, axis=0)
    ue2 = jnp.broadcast_to(ue_row, (NUM_HEADS, D))
    uaa = jnp.concatenate([ue2, area_all], axis=1)
    h = jnp.maximum(
        jnp.dot(uaa, fw1_ref[...], preferred_element_type=jnp.float32)
        + fb1_ref[0:1, :], 0.0)
    o = (jnp.dot(h, fw2_ref[...], preferred_element_type=jnp.float32)
         + fb2_ref[0:1, :])
    m = jnp.max(o, axis=0, keepdims=True)
    e = jnp.exp(o - m)
    s = jnp.sum(e, axis=0, keepdims=True)
    out_ref[...] = jnp.sum((e / s) * area_all, axis=0, keepdims=True)


def kernel(user, input_items, U, I, W1, b1, W2, b2, fW1, fb1, fW2, fb2):
    flat = input_items.reshape(-1).astype(jnp.int32)
    idx = jnp.zeros((ROWS_PAD,), jnp.int32).at[:NUM_HEADS * HIST].set(flat)
    idx8 = idx // 8
    off = idx % 8
    onehot = (off[:, None] ==
              lax.broadcasted_iota(jnp.int32, (ROWS_PAD, 8), 1)
              ).astype(jnp.float32)[:, :, None]
    u32 = jnp.asarray(user, jnp.int32)
    uidx8 = jnp.full((8,), u32 // 8, jnp.int32)
    uoh = (jnp.arange(8, dtype=jnp.int32) == u32 % 8
           ).astype(jnp.float32).reshape(1, 8, 1)

    blocks, ublk = _sc_gather(I.reshape(-1, 8, D), idx8,
                              U.reshape(-1, 8, D), uidx8)

    out = pl.pallas_call(
        _tc_dense_body,
        out_shape=jax.ShapeDtypeStruct((1, D), jnp.float32),
    )(blocks, onehot, ublk, uoh, W1, b1, W2, b2, fW1,
      fb1.reshape(1, 16), fW2, fb2.reshape(1, 1))
    return out


# speedup vs baseline: 2.6462x; 2.6462x over previous
import functools

import jax
import jax.numpy as jnp
from jax import lax
from jax.experimental import pallas as pl
from jax.experimental.pallas import tpu as pltpu
from jax.experimental.pallas import tpu_sc as plsc

NUM_HEADS = 9
HIST = 200
D = 32
NC = 2
NS = 16
NW = NC * NS
ROWS_PAD = 2048
RPW = ROWS_PAD // NW


def _sc_gather(item_blocks, idx_pad, user_blocks, uidx):
    mesh = plsc.VectorSubcoreMesh(
        core_axis_name="c", subcore_axis_name="s", num_cores=NC, num_subcores=NS
    )

    @functools.partial(
        pl.kernel,
        out_type=(
            jax.ShapeDtypeStruct((ROWS_PAD, D), jnp.float32),
            jax.ShapeDtypeStruct((8, D), jnp.float32),
        ),
        mesh=mesh,
        compiler_params=pltpu.CompilerParams(needs_layout_passes=False),
        scratch_types=[
            pltpu.VMEM((RPW,), jnp.int32),
            pltpu.VMEM((RPW, 8, D), jnp.float32),
            pltpu.VMEM((RPW, D), jnp.float32),
            pltpu.VMEM((16,), jnp.int32),
            pltpu.VMEM((8, 8, D), jnp.float32),
            pltpu.VMEM((8, D), jnp.float32),
            pltpu.SemaphoreType.DMA,
            pltpu.SemaphoreType.DMA,
        ],
    )
    def k(items_hbm, idx_hbm, users_hbm, uidx_hbm, out_hbm, uout_hbm,
          idx_v, blocks_v, rows_v, uidx_v, ublk_v, urow_v, sem, usem):
        wid = lax.axis_index("s") * NC + lax.axis_index("c")
        base = wid * RPW
        pltpu.sync_copy(idx_hbm.at[pl.ds(base, RPW)], idx_v)

        cps = []
        chunks = [idx_v[pl.ds(t * 16, 16)] for t in range(RPW // 16)]
        for r in range(RPW):
            blk = chunks[r // 16][r % 16] // 8
            cps.append(pltpu.async_copy(
                items_hbm.at[pl.ds(blk, 1)], blocks_v.at[pl.ds(r, 1)], sem))
        @pl.when(wid == 0)
        def _():
            pltpu.sync_copy(uidx_hbm, uidx_v)
            uchunk = uidx_v[pl.ds(0, 16)]
            ublk = uchunk[0] // 8
            pltpu.async_copy(
                users_hbm.at[pl.ds(ublk, 1)], ublk_v.at[pl.ds(0, 1)],
                usem).wait()
            uoff = jnp.full((16,), uchunk[0] % 8, jnp.int32)
            zero16 = jnp.zeros((16,), jnp.int32)
            lane = lax.iota(jnp.int32, 16)
            for h in range(2):
                v = plsc.load_gather(ublk_v, [zero16, uoff, lane + 16 * h])
                urow_v[0, pl.ds(16 * h, 16)] = v
            pltpu.sync_copy(urow_v, uout_hbm)

        for cp in cps:
            cp.wait()

        lane = lax.iota(jnp.int32, 16)
        for r in range(RPW):
            off = jnp.full((16,), chunks[r // 16][r % 16] % 8, jnp.int32)
            rr = jnp.full((16,), r, jnp.int32)
            for h in range(2):
                v = plsc.load_gather(blocks_v, [rr, off, lane + 16 * h])
                rows_v[r, pl.ds(16 * h, 16)] = v
        pltpu.sync_copy(rows_v, out_hbm.at[pl.ds(base, RPW)])

    return k(item_blocks, idx_pad, user_blocks, uidx)


def _tc_dense_body(gath_ref, urow_ref, w1_ref, b1_ref, w2_ref, b2_ref,
                   fw1_ref, fb1_ref, fw2_ref, fb2_ref, out_ref):
    ue_row = urow_ref[0:1, :]
    ue = jnp.broadcast_to(ue_row, (HIST, D))
    outs = []
    for i in range(NUM_HEADS):
        area = gath_ref[i * HIST:(i + 1) * HIST, :]
        x = jnp.concatenate([ue, area], axis=1)
        h = jnp.maximum(
            jnp.dot(x, w1_ref[i], preferred_element_type=jnp.float32)
            + b1_ref[i:i + 1, :], 0.0)
        o = (jnp.dot(h, w2_ref[i], preferred_element_type=jnp.float32)
             + b2_ref[i:i + 1, :])
        m = jnp.max(o, axis=0, keepdims=True)
        e = jnp.exp(o - m)
        s = jnp.sum(e, axis=0, keepdims=True)
        outs.append(jnp.sum((e / s) * area, axis=0, keepdims=True))
    area_all = jnp.concatenate(outs, axis=0)
    ue2 = jnp.broadcast_to(ue_row, (NUM_HEADS, D))
    uaa = jnp.concatenate([ue2, area_all], axis=1)
    h = jnp.maximum(
        jnp.dot(uaa, fw1_ref[...], preferred_element_type=jnp.float32)
        + fb1_ref[0:1, :], 0.0)
    o = (jnp.dot(h, fw2_ref[...], preferred_element_type=jnp.float32)
         + fb2_ref[0:1, :])
    m = jnp.max(o, axis=0, keepdims=True)
    e = jnp.exp(o - m)
    s = jnp.sum(e, axis=0, keepdims=True)
    out_ref[...] = jnp.sum((e / s) * area_all, axis=0, keepdims=True)


def kernel(user, input_items, U, I, W1, b1, W2, b2, fW1, fb1, fW2, fb2):
    flat = input_items.reshape(-1).astype(jnp.int32)
    idx = jnp.zeros((ROWS_PAD,), jnp.int32).at[:NUM_HEADS * HIST].set(flat)
    uidx = jnp.full((16,), jnp.asarray(user, jnp.int32), jnp.int32)

    gath, urow = _sc_gather(I.reshape(-1, 8, D), idx, U.reshape(-1, 8, D),
                            uidx)

    out = pl.pallas_call(
        _tc_dense_body,
        out_shape=jax.ShapeDtypeStruct((1, D), jnp.float32),
    )(gath, urow, W1, b1, W2, b2, fW1, fb1.reshape(1, 16), fW2,
      fb2.reshape(1, 1))
    return out
